# Initial kernel scaffold; baseline (speedup 1.0000x reference)
import jax
import jax.numpy as jnp
from jax.experimental import pallas as pl


def kernel(x, edge_index, res_W, res_b, c1_W, c1_as, c1_ad, c1_b, bn1_g, bn1_b, bn1_m, bn1_v, c2_W, c2_as, c2_ad, c2_b, bn2_g, bn2_b, bn2_m, bn2_v, f1_W, f1_b, fbn_g, fbn_b, fbn_m, fbn_v, f2_W, f2_b):
    raise NotImplementedError("write your pallas kernel here")


# speedup vs baseline: 9.2400x; 9.2400x over previous
import jax
import jax.numpy as jnp
from jax import lax
from jax.experimental import pallas as pl
from jax.experimental.pallas import tpu as pltpu
from jax.experimental.pallas import tpu_sc as plsc

N = 10000
IN_SIZE = 128
HEADS = 8
HEAD_DIM = 32
HID = 256
OUT_SIZE = 40
E = 320000

NP = 10112
RB = 128
NBLK = NP // RB

NE1 = E + N
EP1 = 331776
NE2 = NE1 + N
EP2 = 342016
K_TOP = int(NE1 * 0.3)
KSEL = K_TOP + N

NSUB = 16
CPT1 = EP1 // NSUB
CPT2 = EP2 // NSUB
RPT = NP // NSUB

BE1 = 128
BE2 = 64

f32 = jnp.float32
i32 = jnp.int32


def _bn(x, g, b, m, v):
    return (x - m) * jax.lax.rsqrt(v + 1e-5) * g + b


def _elu(x):
    return jnp.where(x > 0, x, jnp.exp(jnp.minimum(x, 0.0)) - 1.0)


def _tc1_body(x_ref, rw_ref, rb_ref, c1w_ref, am_ref,
              xp_ref, xl_ref, as_ref, ad_ref):
    xp = jnp.dot(x_ref[...], rw_ref[...], preferred_element_type=f32)
    xp = xp + rb_ref[...]
    xp_ref[...] = xp
    xl = jnp.dot(xp, c1w_ref[...], preferred_element_type=f32)
    xl_ref[0] = xl[:, :128]
    xl_ref[1] = xl[:, 128:]
    a = jnp.dot(xl, am_ref[...], preferred_element_type=f32)
    as_ref[...] = a[:, :16]
    ad_ref[...] = a[:, 16:32]


def _tc2_body(m_ref, pv_ref, c2w_ref, am_ref,
              xl_ref, as_ref, ad_ref):
    msg = jnp.concatenate([m_ref[0], m_ref[1]], axis=1)
    pv = pv_ref[...]
    h1 = msg + pv[0:1, :]
    h = _elu(_bn(h1, pv[1:2, :], pv[2:3, :], pv[3:4, :], pv[4:5, :]))
    xl = jnp.dot(h, c2w_ref[...], preferred_element_type=f32)
    xl_ref[0] = xl[:, :128]
    xl_ref[1] = xl[:, 128:]
    a = jnp.dot(xl, am_ref[...], preferred_element_type=f32)
    as_ref[...] = a[:, :16]
    ad_ref[...] = a[:, 16:32]


def _tc3_body(m_ref, pv_ref, xp_ref, f1w_ref, q_ref, f2w_ref,
              out_ref):
    msg = jnp.concatenate([m_ref[0], m_ref[1]], axis=1)
    pv = pv_ref[...]
    h1 = msg + pv[0:1, :]
    h2 = _elu(_bn(h1, pv[1:2, :], pv[2:3, :], pv[3:4, :], pv[4:5, :]))
    h2 = h2 + xp_ref[...]
    z = jnp.dot(h2, f1w_ref[...], preferred_element_type=f32)
    q = q_ref[...]
    z = z + q[0:1, :]
    z = jnp.maximum(_bn(z, q[1:2, :], q[2:3, :], q[3:4, :], q[4:5, :]), 0.0)
    lg = jnp.dot(z, f2w_ref[...], preferred_element_type=f32)
    lg = lg + q[5:6, :]
    col = lax.broadcasted_iota(i32, lg.shape, 1)
    lgm = jnp.where(col < OUT_SIZE, lg, -1e30)
    mx = jnp.max(lgm, axis=1, keepdims=True)
    ex = jnp.where(col < OUT_SIZE, jnp.exp(lg - mx), 0.0)
    s = jnp.sum(ex, axis=1, keepdims=True)
    out_ref[...] = lg - mx - jnp.log(s)


def _row_spec(w):
    return pl.BlockSpec((RB, w), lambda i: (i, 0))


def _full_spec(shape):
    nd = len(shape)
    return pl.BlockSpec(shape, lambda i: (0,) * nd)


def _zero_vmem(ref, rows, width):
    z = jnp.zeros((16,), f32)

    def body(j, _):
        for v in range(width // 16):
            ref[j, pl.ds(16 * v, 16)] = z
        return 0

    lax.fori_loop(0, rows, body, 0)


def _zero_shared_stripe(zb_ref, chunk, sh_ref, row0, rows):
    nfull = rows // chunk
    rem = rows - nfull * chunk
    for i in range(nfull):
        pltpu.sync_copy(zb_ref, sh_ref.at[pl.ds(row0 + i * chunk, chunk)])
    if rem:
        pltpu.sync_copy(zb_ref.at[pl.ds(0, rem)],
                        sh_ref.at[pl.ds(row0 + nfull * chunk, rem)])


def _edge_alpha_num(ea_s_ref, ea_d_ref, j, mask8):
    va = ea_s_ref[j] + ea_d_ref[j]
    va = jnp.where(va >= 0.0, va, 0.2 * va)
    ve = jnp.exp(va)
    return jnp.where(mask8, ve, 0.0)


def _sc_conv(be, src_h, dst_h, asr_h, adr_h, xl_h, selm_src, n_blocks,
             write_scores, scores_out,
             sidx, didx, ea_s, ea_d, exb, esum, alf, msg, scr,
             sums_sh, acc_sh, sem, c, base0, mask8, lane):
    def p1(b, _):
        eb = pl.multiple_of(base0 + b * be, be)
        pltpu.sync_copy(src_h.at[pl.ds(eb, be)], sidx)
        pltpu.sync_copy(dst_h.at[pl.ds(eb, be)], didx)
        pltpu.async_copy(asr_h.at[sidx], ea_s, sem).wait()
        pltpu.async_copy(adr_h.at[didx], ea_d, sem).wait()

        def rows(j, _):
            ve = _edge_alpha_num(ea_s, ea_d, j, mask8)
            if selm_src is not None:
                ve = ve * selm_src(b * be + j)
            exb[j] = ve
            return 0

        lax.fori_loop(0, be, rows, 0)
        pltpu.sync_copy(exb, sums_sh.at[didx], add=True)
        return 0

    lax.fori_loop(0, n_blocks, p1, 0)
    plsc.subcore_barrier()

    xl_src = xl_h.at[c]

    def p2(b, _):
        eb = pl.multiple_of(base0 + b * be, be)
        pltpu.sync_copy(src_h.at[pl.ds(eb, be)], sidx)
        pltpu.sync_copy(dst_h.at[pl.ds(eb, be)], didx)
        pltpu.async_copy(asr_h.at[sidx], ea_s, sem).wait()
        pltpu.async_copy(adr_h.at[didx], ea_d, sem).wait()
        pltpu.async_copy(sums_sh.at[didx], esum, sem).wait()
        pltpu.async_copy(xl_src.at[sidx], msg, sem).wait()

        def rows(j, _):
            ve = _edge_alpha_num(ea_s, ea_d, j, mask8)
            if selm_src is not None:
                ve = ve * selm_src(b * be + j)
            al = ve / (esum[j] + 1e-16)
            alf[pl.ds(pl.multiple_of(j * 16, 16), 16)] = al
            jv = jnp.full((16,), j * 16, i32)
            for v in range(8):
                hv = c * 4 + (v // 2)
                am = plsc.load_gather(alf, [jv + hv])
                sl = pl.ds(16 * v, 16)
                msg[j, sl] = msg[j, sl] * am
            return 0

        lax.fori_loop(0, be, rows, 0)
        pltpu.sync_copy(msg, acc_sh.at[didx], add=True)

        if write_scores:
            @pl.when(c == 0)
            def _():
                for j16 in range(be // 16):
                    rv = (jnp.full((16,), j16 * 16, i32) + lane) * 16
                    acc = jnp.zeros((16,), f32)
                    for h in range(8):
                        acc = acc + plsc.load_gather(alf, [rv + h])
                    sc = acc * 0.125
                    eid = jnp.full((16,), eb + j16 * 16, i32) + lane
                    sc = jnp.where(eid < NE1, sc, -1.0)
                    scr[pl.ds(j16 * 16, 16)] = sc
                pltpu.sync_copy(scr, scores_out.at[pl.ds(eb, be)])
        return 0

    lax.fori_loop(0, n_blocks, p2, 0)
    plsc.subcore_barrier()


def _write_out_half(msgs_out, acc_sh, c, row0, chunk):
    out_half = msgs_out.at[c]
    nfull = RPT // chunk
    rem = RPT - nfull * chunk
    for i in range(nfull):
        pltpu.sync_copy(acc_sh.at[pl.ds(row0 + i * chunk, chunk)],
                        out_half.at[pl.ds(row0 + i * chunk, chunk)])
    if rem:
        pltpu.sync_copy(acc_sh.at[pl.ds(row0 + RPT - rem, rem)],
                        out_half.at[pl.ds(row0 + RPT - rem, rem)])


def _sc1_body(src_h, dst_h, asr_h, adr_h, xl_h,
              scores_out, msgs_out,
              sidx, didx, ea_s, ea_d, exb, esum, alf, msg, scr,
              acc_sh, sums_sh, sem):
    c = lax.axis_index("c")
    s = lax.axis_index("s")
    lane = lax.iota(i32, 16)
    mask8 = lane < 8
    row0 = pl.multiple_of(s * RPT, 8)

    _zero_vmem(msg, BE1, 128)
    _zero_shared_stripe(msg, BE1, acc_sh, row0, RPT)
    _zero_vmem(exb, BE1, 16)
    _zero_shared_stripe(exb, BE1, sums_sh, row0, RPT)
    plsc.subcore_barrier()

    base0 = pl.multiple_of(s * CPT1, BE1)
    _sc_conv(BE1, src_h, dst_h, asr_h, adr_h, xl_h, None, CPT1 // BE1,
             True, scores_out,
             sidx, didx, ea_s, ea_d, exb, esum, alf, msg, scr,
             sums_sh, acc_sh, sem, c, base0, mask8, lane)

    _write_out_half(msgs_out, acc_sh, c, row0, BE1)


def _sc2_body(src_h, dst_h, asr_h, adr_h, xl_h, scores2_h,
              msgs_out,
              sidx, didx, ea_s, ea_d, exb, esum, alf, msg,
              keys, cb, stb,
              acc_sh, sums_sh, stage_sh, stage2_sh, sem):
    c = lax.axis_index("c")
    s = lax.axis_index("s")
    lane = lax.iota(i32, 16)
    mask8 = lane < 8
    row0 = pl.multiple_of(s * RPT, 8)
    base0 = pl.multiple_of(s * CPT2, BE2)

    _zero_vmem(msg, BE2, 128)
    _zero_shared_stripe(msg, BE2, acc_sh, row0, RPT)
    _zero_vmem(exb, BE2, 16)
    _zero_shared_stripe(exb, BE2, sums_sh, row0, RPT)
    pltpu.sync_copy(scores2_h.at[pl.ds(base0, CPT2)], keys)
    plsc.subcore_barrier()

    nv = CPT2 // 16

    def count_gt(thr):
        def cbody(v, acc):
            kf = keys[pl.ds(16 * v, 16)]
            ki = plsc.bitcast(kf, i32)
            return acc + jnp.where(ki > thr, 1, 0).astype(i32)
        acc = lax.fori_loop(0, nv, cbody, jnp.zeros((16,), i32))
        return jnp.sum(acc)

    def stage_scalar(val, sh):
        cb[...] = jnp.full((16,), val, i32)
        pltpu.sync_copy(cb, sh.at[s])

    def read_total(sh):
        pltpu.sync_copy(sh, stb)
        tot = jnp.zeros((16,), i32)
        for r in range(16):
            tot = tot + stb[r]
        return tot[0]

    def wcond(carry):
        lo, hi = carry
        return hi - lo > 1

    def wbody(carry):
        lo, hi = carry
        mid = (lo + hi) // 2
        cnt = count_gt(mid)
        stage_scalar(cnt, stage_sh)
        plsc.subcore_barrier()
        tot = read_total(stage_sh)
        plsc.subcore_barrier()
        pred = tot >= KSEL
        return (jnp.where(pred, mid, lo), jnp.where(pred, hi, mid))

    lo0 = jnp.asarray(-2, i32)
    hi0 = jnp.asarray(1 << 30, i32)
    _, thr = lax.while_loop(wcond, wbody, (lo0, hi0))

    def gq_body(v, acc):
        g, q = acc
        kf = keys[pl.ds(16 * v, 16)]
        ki = plsc.bitcast(kf, i32)
        g = g + jnp.where(ki > thr, 1, 0).astype(i32)
        q = q + jnp.where(ki == thr, 1, 0).astype(i32)
        return (g, q)

    gv, qv = lax.fori_loop(0, nv, gq_body,
                           (jnp.zeros((16,), i32), jnp.zeros((16,), i32)))
    stage_scalar(jnp.sum(gv), stage_sh)
    stage_scalar(jnp.sum(qv), stage2_sh)
    plsc.subcore_barrier()
    g_tot = read_total(stage_sh)
    pltpu.sync_copy(stage2_sh, stb)
    pref = jnp.asarray(0, i32)
    for r in range(16):
        qr = stb[r][0]
        pref = pref + jnp.where(r < s, qr, 0)
    plsc.subcore_barrier()
    need = KSEL - g_tot

    def sel_body(v, cnt):
        kf = keys[pl.ds(16 * v, 16)]
        ki = plsc.bitcast(kf, i32)
        gt = ki > thr
        eq = ki == thr
        eqi = jnp.where(eq, 1, 0).astype(i32)
        pre = plsc.cumsum(eqi)
        rank = jnp.full((16,), cnt, i32) + pre - 1
        keep = jnp.logical_or(gt, jnp.logical_and(eq, rank < need))
        keys[pl.ds(16 * v, 16)] = jnp.where(keep, 1.0, 0.0)
        return cnt + jnp.sum(eqi)

    lax.fori_loop(0, nv, sel_body, pref)
    plsc.subcore_barrier()

    def selv(j_local):
        return plsc.load_gather(keys, [jnp.full((16,), j_local, i32)])

    _sc_conv(BE2, src_h, dst_h, asr_h, adr_h, xl_h, selv, CPT2 // BE2,
             False, None,
             sidx, didx, ea_s, ea_d, exb, esum, alf, msg, None,
             sums_sh, acc_sh, sem, c, base0, mask8, lane)

    _write_out_half(msgs_out, acc_sh, c, row0, BE2)


_SC_MESH = plsc.VectorSubcoreMesh(core_axis_name="c", subcore_axis_name="s")


def _conv_scratch(be):
    return [
        pltpu.VMEM((be,), i32),
        pltpu.VMEM((be,), i32),
        pltpu.VMEM((be, 16), f32),
        pltpu.VMEM((be, 16), f32),
        pltpu.VMEM((be, 16), f32),
        pltpu.VMEM((be, 16), f32),
        pltpu.VMEM((be * 16,), f32),
        pltpu.VMEM((be, 128), f32),
    ]


_SC_PARAMS = pltpu.CompilerParams(needs_layout_passes=False,
                                  use_tc_tiling_on_sc=False)

_sc1_call = pl.kernel(
    _sc1_body,
    out_type=(jax.ShapeDtypeStruct((EP1,), f32),
              jax.ShapeDtypeStruct((2, NP, 128), f32)),
    mesh=_SC_MESH,
    compiler_params=_SC_PARAMS,
    scratch_types=_conv_scratch(BE1) + [
        pltpu.VMEM((BE1,), f32),
        pltpu.VMEM_SHARED((NP, 128), f32),
        pltpu.VMEM_SHARED((NP, 16), f32),
        pltpu.SemaphoreType.DMA,
    ],
)

_sc2_call = pl.kernel(
    _sc2_body,
    out_type=jax.ShapeDtypeStruct((2, NP, 128), f32),
    mesh=_SC_MESH,
    compiler_params=_SC_PARAMS,
    scratch_types=_conv_scratch(BE2) + [
        pltpu.VMEM((CPT2,), f32),
        pltpu.VMEM((16,), i32),
        pltpu.VMEM((16, 16), i32),
        pltpu.VMEM_SHARED((NP, 128), f32),
        pltpu.VMEM_SHARED((NP, 16), f32),
        pltpu.VMEM_SHARED((16, 16), i32),
        pltpu.VMEM_SHARED((16, 16), i32),
        pltpu.SemaphoreType.DMA,
    ],
)


def _att_mat(att_s, att_d):
    fl_s = att_s.reshape(HID)
    fl_d = att_d.reshape(HID)
    fidx = jnp.arange(HID)
    head = fidx // HEAD_DIM
    m = jnp.zeros((HID, 32), f32)
    m = m.at[fidx, head].set(fl_s)
    m = m.at[fidx, 16 + head].set(fl_d)
    return m


def kernel(x, edge_index, res_W, res_b, c1_W, c1_as, c1_ad, c1_b,
           bn1_g, bn1_b, bn1_m, bn1_v, c2_W, c2_as, c2_ad, c2_b,
           bn2_g, bn2_b, bn2_m, bn2_v, f1_W, f1_b,
           fbn_g, fbn_b, fbn_m, fbn_v, f2_W, f2_b):
    xpad = jnp.zeros((NP, IN_SIZE), f32).at[:N].set(x)

    loops = jnp.arange(N, dtype=i32)
    src0 = edge_index[0].astype(i32)
    dst0 = edge_index[1].astype(i32)
    padN = jnp.full((EP1 - NE1,), N, i32)
    src1 = jnp.concatenate([src0, loops, padN])
    dst1 = jnp.concatenate([dst0, loops, padN])
    padN2 = jnp.full((EP2 - NE2,), N, i32)
    src2 = jnp.concatenate([src0, loops, loops, padN2])
    dst2 = jnp.concatenate([dst0, loops, loops, padN2])

    am1 = _att_mat(c1_as, c1_ad)
    am2 = _att_mat(c2_as, c2_ad)

    xp, xl1, as1, ad1 = pl.pallas_call(
        _tc1_body,
        grid=(NBLK,),
        in_specs=[_row_spec(IN_SIZE), _full_spec((IN_SIZE, HID)),
                  _full_spec((1, HID)), _full_spec((HID, HID)),
                  _full_spec((HID, 32))],
        out_specs=[_row_spec(HID),
                   pl.BlockSpec((2, RB, 128), lambda i: (0, i, 0)),
                   _row_spec(16), _row_spec(16)],
        out_shape=[jax.ShapeDtypeStruct((NP, HID), f32),
                   jax.ShapeDtypeStruct((2, NP, 128), f32),
                   jax.ShapeDtypeStruct((NP, 16), f32),
                   jax.ShapeDtypeStruct((NP, 16), f32)],
    )(xpad, res_W, res_b.reshape(1, HID), c1_W, am1)

    scores1, msgs1 = _sc1_call(src1, dst1, as1, ad1, xl1)

    pv1 = jnp.stack([c1_b, bn1_g, bn1_b, bn1_m, bn1_v])
    xl2, as2, ad2 = pl.pallas_call(
        _tc2_body,
        grid=(NBLK,),
        in_specs=[pl.BlockSpec((2, RB, 128), lambda i: (0, i, 0)),
                  _full_spec((5, HID)), _full_spec((HID, HID)),
                  _full_spec((HID, 32))],
        out_specs=[pl.BlockSpec((2, RB, 128), lambda i: (0, i, 0)),
                   _row_spec(16), _row_spec(16)],
        out_shape=[jax.ShapeDtypeStruct((2, NP, 128), f32),
                   jax.ShapeDtypeStruct((NP, 16), f32),
                   jax.ShapeDtypeStruct((NP, 16), f32)],
    )(msgs1, pv1, c2_W, am2)

    scores2 = jnp.concatenate([
        scores1[:NE1],
        jnp.full((N,), 2.0, f32),
        jnp.full((EP2 - NE2,), -1.0, f32)
    ])
    msgs2 = _sc2_call(src2, dst2, as2, ad2, xl2, scores2)

    pv2 = jnp.stack([c2_b, bn2_g, bn2_b, bn2_m, bn2_v])
    f2w_pad = jnp.zeros((HID // 2, 128), f32).at[:, :OUT_SIZE].set(f2_W)
    qv = jnp.stack([f1_b, fbn_g, fbn_b, fbn_m, fbn_v,
                    jnp.zeros((HID // 2,), f32)])
    qv = jnp.zeros((6, 128), f32).at[:, :HID // 2].set(qv)
    qv = qv.at[5, :OUT_SIZE].set(f2_b)
    out = pl.pallas_call(
        _tc3_body,
        grid=(NBLK,),
        in_specs=[pl.BlockSpec((2, RB, 128), lambda i: (0, i, 0)),
                  _full_spec((5, HID)), _row_spec(HID),
                  _full_spec((HID, HID // 2)), _full_spec((6, 128)),
                  _full_spec((HID // 2, 128))],
        out_specs=[_row_spec(128)],
        out_shape=[jax.ShapeDtypeStruct((NP, 128), f32)],
    )(msgs2, pv2, xp, f1_W, qv, f2w_pad)[0]

    return out[:N, :OUT_SIZE]


# speedup vs baseline: 13.7277x; 1.4857x over previous
import jax
import jax.numpy as jnp
from jax import lax
from jax.experimental import pallas as pl
from jax.experimental.pallas import tpu as pltpu
from jax.experimental.pallas import tpu_sc as plsc

N = 10000
IN_SIZE = 128
HEADS = 8
HEAD_DIM = 32
HID = 256
OUT_SIZE = 40
E = 320000

NP = 10112
RB = 128
NBLK = NP // RB

NE1 = E + N
EP1 = 331776
NE2 = NE1 + N
EP2 = 344064
K_TOP = int(NE1 * 0.3)
KSEL = K_TOP + N

NSUB = 16
CPT1 = EP1 // NSUB
CPT2 = EP2 // NSUB
RPT = NP // NSUB

BE1 = 64
BPR1 = 12
NRND1 = CPT1 // (BE1 * BPR1)
BE2 = 32
BPR2 = 12
NRND2 = CPT2 // (BE2 * BPR2)

f32 = jnp.float32
i32 = jnp.int32


def _bn(x, g, b, m, v):
    return (x - m) * jax.lax.rsqrt(v + 1e-5) * g + b


def _elu(x):
    return jnp.where(x > 0, x, jnp.exp(jnp.minimum(x, 0.0)) - 1.0)


def _tc1_body(x_ref, rw_ref, rb_ref, c1w_ref, am_ref,
              xp_ref, xl_ref, as_ref, ad_ref):
    xp = jnp.dot(x_ref[...], rw_ref[...], preferred_element_type=f32)
    xp = xp + rb_ref[...]
    xp_ref[...] = xp
    xl = jnp.dot(xp, c1w_ref[...], preferred_element_type=f32)
    xl_ref[0] = xl[:, :128]
    xl_ref[1] = xl[:, 128:]
    a = jnp.dot(xl, am_ref[...], preferred_element_type=f32)
    as_ref[...] = a[:, :16]
    ad_ref[...] = a[:, 16:32]


def _tc2_body(m_ref, pv_ref, c2w_ref, am_ref,
              xl_ref, as_ref, ad_ref):
    msg = jnp.concatenate([m_ref[0], m_ref[1]], axis=1)
    pv = pv_ref[...]
    h1 = msg + pv[0:1, :]
    h = _elu(_bn(h1, pv[1:2, :], pv[2:3, :], pv[3:4, :], pv[4:5, :]))
    xl = jnp.dot(h, c2w_ref[...], preferred_element_type=f32)
    xl_ref[0] = xl[:, :128]
    xl_ref[1] = xl[:, 128:]
    a = jnp.dot(xl, am_ref[...], preferred_element_type=f32)
    as_ref[...] = a[:, :16]
    ad_ref[...] = a[:, 16:32]


def _tc3_body(m_ref, pv_ref, xp_ref, f1w_ref, q_ref, f2w_ref,
              out_ref):
    msg = jnp.concatenate([m_ref[0], m_ref[1]], axis=1)
    pv = pv_ref[...]
    h1 = msg + pv[0:1, :]
    h2 = _elu(_bn(h1, pv[1:2, :], pv[2:3, :], pv[3:4, :], pv[4:5, :]))
    h2 = h2 + xp_ref[...]
    z = jnp.dot(h2, f1w_ref[...], preferred_element_type=f32)
    q = q_ref[...]
    z = z + q[0:1, :]
    z = jnp.maximum(_bn(z, q[1:2, :], q[2:3, :], q[3:4, :], q[4:5, :]), 0.0)
    lg = jnp.dot(z, f2w_ref[...], preferred_element_type=f32)
    lg = lg + q[5:6, :]
    col = lax.broadcasted_iota(i32, lg.shape, 1)
    lgm = jnp.where(col < OUT_SIZE, lg, -1e30)
    mx = jnp.max(lgm, axis=1, keepdims=True)
    ex = jnp.where(col < OUT_SIZE, jnp.exp(lg - mx), 0.0)
    s = jnp.sum(ex, axis=1, keepdims=True)
    out_ref[...] = lg - mx - jnp.log(s)


def _row_spec(w):
    return pl.BlockSpec((RB, w), lambda i: (i, 0))


def _full_spec(shape):
    nd = len(shape)
    return pl.BlockSpec(shape, lambda i: (0,) * nd)


def _zero_vmem(ref, rows, width):
    z = jnp.zeros((16,), f32)

    def body(j, _):
        for v in range(width // 16):
            ref[j, pl.ds(16 * v, 16)] = z
        return 0

    lax.fori_loop(0, rows, body, 0)


def _zero_shared_stripe(zb_ref, chunk, sh_ref, row0, rows):
    nfull = rows // chunk
    rem = rows - nfull * chunk
    for i in range(nfull):
        pltpu.sync_copy(zb_ref, sh_ref.at[pl.ds(row0 + i * chunk, chunk)])
    if rem:
        pltpu.sync_copy(zb_ref.at[pl.ds(0, rem)],
                        sh_ref.at[pl.ds(row0 + nfull * chunk, rem)])


def _edge_alpha_num(ea_s_ref, ea_d_ref, j, mask8):
    va = ea_s_ref[j] + ea_d_ref[j]
    va = jnp.where(va >= 0.0, va, 0.2 * va)
    ve = jnp.exp(va)
    return jnp.where(mask8, ve, 0.0)


def _pipelined_blocks(bpr, fire, compute, sfire):
    handles = [None, None]
    handles[0] = fire(0, 0)
    for b in range(bpr):
        cur = b % 2
        nxt = 1 - cur
        if b + 1 < bpr:
            handles[nxt] = fire(b + 1, nxt)
        for d in handles[cur]:
            d.wait()
        compute(b, cur)
        sfire(b, cur)


def _sc_conv(be, bpr, nrnd, src2d_h, dst2d_h, asr_h, adr_h, xl_h,
             selm_src, write_scores, scores_out,
             sslab, dslab, sidx, didx, ea_s, ea_d, exb, esum, alf, msg, scr,
             sums_sh, acc_sh, semg, sems, c, s, mask8, lane):
    rbase0 = s * (bpr * nrnd)
    ebase0 = s * (be * bpr * nrnd)

    def load_slabs(r):
        row = pl.multiple_of(rbase0 + r * bpr, 4)
        pltpu.sync_copy(src2d_h.at[pl.ds(row, bpr)], sslab)
        pltpu.sync_copy(dst2d_h.at[pl.ds(row, bpr)], dslab)

    def eb_of(r, b):
        return pl.multiple_of(ebase0 + (r * bpr + b) * be, be)

    def p1_round(r, _):
        load_slabs(r)

        def fire(b, i):
            for v in range(be // 16):
                sidx[i][pl.ds(16 * v, 16)] = sslab[b, pl.ds(16 * v, 16)]
                didx[i][pl.ds(16 * v, 16)] = dslab[b, pl.ds(16 * v, 16)]
            return [
                pltpu.async_copy(asr_h.at[sidx[i]], ea_s[i], semg[i]),
                pltpu.async_copy(adr_h.at[didx[i]], ea_d[i], semg[i]),
            ]

        def compute(b, i):
            loc0 = (r * bpr + b) * be

            def rows(j, _):
                ve = _edge_alpha_num(ea_s[i], ea_d[i], j, mask8)
                if selm_src is not None:
                    ve = ve * selm_src(loc0 + j)
                exb[i][j] = ve
                return 0

            lax.fori_loop(0, be, rows, 0)

        def sfire(b, i):
            pltpu.sync_copy(exb[i], sums_sh.at[didx[i]], add=True)

        _pipelined_blocks(bpr, fire, compute, sfire)
        return 0

    lax.fori_loop(0, nrnd, p1_round, 0)
    plsc.subcore_barrier()

    xl_src = xl_h.at[c]

    def p2_round(r, _):
        load_slabs(r)

        def fire(b, i):
            for v in range(be // 16):
                sidx[i][pl.ds(16 * v, 16)] = sslab[b, pl.ds(16 * v, 16)]
                didx[i][pl.ds(16 * v, 16)] = dslab[b, pl.ds(16 * v, 16)]
            return [
                pltpu.async_copy(asr_h.at[sidx[i]], ea_s[i], semg[i]),
                pltpu.async_copy(adr_h.at[didx[i]], ea_d[i], semg[i]),
                pltpu.async_copy(xl_src.at[sidx[i]], msg[i], semg[i]),
            ]

        def compute(b, i):
            pltpu.async_copy(sums_sh.at[didx[i]], esum[i], semg[i]).wait()
            loc0 = (r * bpr + b) * be

            def rows(j, _):
                ve = _edge_alpha_num(ea_s[i], ea_d[i], j, mask8)
                if selm_src is not None:
                    ve = ve * selm_src(loc0 + j)
                al = ve / (esum[i][j] + 1e-16)
                alf[pl.ds(pl.multiple_of(j * 16, 16), 16)] = al
                jv = jnp.full((16,), j * 16, i32)
                for v in range(8):
                    hv = c * 4 + (v // 2)
                    am = plsc.load_gather(alf, [jv + hv])
                    sl = pl.ds(16 * v, 16)
                    msg[i][j, sl] = msg[i][j, sl] * am
                return 0

            lax.fori_loop(0, be, rows, 0)

            if write_scores:
                @pl.when(c == 0)
                def _():
                    eb = eb_of(r, b)
                    for j16 in range(be // 16):
                        rv = (jnp.full((16,), j16 * 16, i32) + lane) * 16
                        acc = jnp.zeros((16,), f32)
                        for h in range(8):
                            acc = acc + plsc.load_gather(alf, [rv + h])
                        sc = acc * 0.125
                        eid = jnp.full((16,), eb + j16 * 16, i32) + lane
                        sc = jnp.where(eid < NE1, sc, -1.0)
                        scr[pl.ds(j16 * 16, 16)] = sc
                    pltpu.sync_copy(scr, scores_out.at[pl.ds(eb, be)])

        def sfire(b, i):
            pltpu.sync_copy(msg[i], acc_sh.at[didx[i]], add=True)

        def sdrain(i):
            pass

        _pipelined_blocks(bpr, fire, compute, sfire)
        return 0

    lax.fori_loop(0, nrnd, p2_round, 0)
    plsc.subcore_barrier()


def _write_out_half(msgs_out, acc_sh, c, row0, chunk):
    out_half = msgs_out.at[c]
    nfull = RPT // chunk
    rem = RPT - nfull * chunk
    for i in range(nfull):
        pltpu.sync_copy(acc_sh.at[pl.ds(row0 + i * chunk, chunk)],
                        out_half.at[pl.ds(row0 + i * chunk, chunk)])
    if rem:
        pltpu.sync_copy(acc_sh.at[pl.ds(row0 + RPT - rem, rem)],
                        out_half.at[pl.ds(row0 + RPT - rem, rem)])


def _sc1_body(src2d_h, dst2d_h, asr_h, adr_h, xl_h,
              scores_out, msgs_out,
              sslab, dslab, sidxA, sidxB, didxA, didxB,
              ea_sA, ea_sB, ea_dA, ea_dB,
              exbA, exbB, esumA, esumB, alf, msgA, msgB, scr,
              acc_sh, sums_sh, semgA, semgB, semsA, semsB):
    c = lax.axis_index("c")
    s = lax.axis_index("s")
    lane = lax.iota(i32, 16)
    mask8 = lane < 8
    row0 = pl.multiple_of(s * RPT, 8)

    _zero_vmem(msgA, BE1, 128)
    _zero_shared_stripe(msgA, BE1, acc_sh, row0, RPT)
    _zero_vmem(exbA, BE1, 16)
    _zero_shared_stripe(exbA, BE1, sums_sh, row0, RPT)
    plsc.subcore_barrier()

    _sc_conv(BE1, BPR1, NRND1, src2d_h, dst2d_h, asr_h, adr_h,
             xl_h, None, True, scores_out,
             sslab, dslab, [sidxA, sidxB], [didxA, didxB],
             [ea_sA, ea_sB], [ea_dA, ea_dB],
             [exbA, exbB], [esumA, esumB], alf, [msgA, msgB], scr,
             sums_sh, acc_sh, [semgA, semgB], [semsA, semsB],
             c, s, mask8, lane)

    _write_out_half(msgs_out, acc_sh, c, row0, BE1)


def _sc2_body(src2d_h, dst2d_h, asr_h, adr_h, xl_h, scores2_h,
              msgs_out,
              sslab, dslab, sidxA, sidxB, didxA, didxB,
              ea_sA, ea_sB, ea_dA, ea_dB,
              exbA, exbB, esumA, esumB, alf, msgA, msgB,
              keys, cb, stb,
              acc_sh, sums_sh, stage_sh, stage2_sh,
              semgA, semgB, semsA, semsB):
    c = lax.axis_index("c")
    s = lax.axis_index("s")
    lane = lax.iota(i32, 16)
    mask8 = lane < 8
    row0 = pl.multiple_of(s * RPT, 8)
    base0 = pl.multiple_of(s * CPT2, 8)

    _zero_vmem(msgA, BE2, 128)
    _zero_shared_stripe(msgA, BE2, acc_sh, row0, RPT)
    _zero_vmem(exbA, BE2, 16)
    _zero_shared_stripe(exbA, BE2, sums_sh, row0, RPT)
    pltpu.sync_copy(scores2_h.at[pl.ds(base0, CPT2)], keys)
    plsc.subcore_barrier()

    nv = CPT2 // 16

    def count_gt(thr):
        def cbody(v, acc):
            kf = keys[pl.ds(16 * v, 16)]
            ki = plsc.bitcast(kf, i32)
            return acc + jnp.where(ki > thr, 1, 0).astype(i32)
        acc = lax.fori_loop(0, nv, cbody, jnp.zeros((16,), i32))
        return jnp.sum(acc)

    def stage_scalar(val, sh):
        cb[...] = jnp.full((16,), val, i32)
        pltpu.sync_copy(cb, sh.at[s])

    def read_total(sh):
        pltpu.sync_copy(sh, stb)
        tot = jnp.zeros((16,), i32)
        for r in range(16):
            tot = tot + stb[r]
        return tot[0]

    def wcond(carry):
        lo, hi = carry
        return hi - lo > 1

    def wbody(carry):
        lo, hi = carry
        mid = (lo + hi) // 2
        cnt = count_gt(mid)
        stage_scalar(cnt, stage_sh)
        plsc.subcore_barrier()
        tot = read_total(stage_sh)
        plsc.subcore_barrier()
        pred = tot >= KSEL
        return (jnp.where(pred, mid, lo), jnp.where(pred, hi, mid))

    lo0 = jnp.asarray(-2, i32)
    hi0 = jnp.asarray(1 << 30, i32)
    _, thr = lax.while_loop(wcond, wbody, (lo0, hi0))

    def gq_body(v, acc):
        g, q = acc
        kf = keys[pl.ds(16 * v, 16)]
        ki = plsc.bitcast(kf, i32)
        g = g + jnp.where(ki > thr, 1, 0).astype(i32)
        q = q + jnp.where(ki == thr, 1, 0).astype(i32)
        return (g, q)

    gv, qv = lax.fori_loop(0, nv, gq_body,
                           (jnp.zeros((16,), i32), jnp.zeros((16,), i32)))
    stage_scalar(jnp.sum(gv), stage_sh)
    stage_scalar(jnp.sum(qv), stage2_sh)
    plsc.subcore_barrier()
    g_tot = read_total(stage_sh)
    pltpu.sync_copy(stage2_sh, stb)
    pref = jnp.asarray(0, i32)
    for r in range(16):
        qr = stb[r][0]
        pref = pref + jnp.where(r < s, qr, 0)
    plsc.subcore_barrier()
    need = KSEL - g_tot

    def sel_body(v, cnt):
        kf = keys[pl.ds(16 * v, 16)]
        ki = plsc.bitcast(kf, i32)
        gt = ki > thr
        eq = ki == thr
        eqi = jnp.where(eq, 1, 0).astype(i32)
        pre = plsc.cumsum(eqi)
        rank = jnp.full((16,), cnt, i32) + pre - 1
        keep = jnp.logical_or(gt, jnp.logical_and(eq, rank < need))
        keys[pl.ds(16 * v, 16)] = jnp.where(keep, 1.0, 0.0)
        return cnt + jnp.sum(eqi)

    lax.fori_loop(0, nv, sel_body, pref)
    plsc.subcore_barrier()

    def selv(loc):
        return plsc.load_gather(keys, [jnp.full((16,), loc, i32)])

    _sc_conv(BE2, BPR2, NRND2, src2d_h, dst2d_h, asr_h, adr_h,
             xl_h, selv, False, None,
             sslab, dslab, [sidxA, sidxB], [didxA, didxB],
             [ea_sA, ea_sB], [ea_dA, ea_dB],
             [exbA, exbB], [esumA, esumB], alf, [msgA, msgB], None,
             sums_sh, acc_sh, [semgA, semgB], [semsA, semsB],
             c, s, mask8, lane)

    _write_out_half(msgs_out, acc_sh, c, row0, BE2)


_SC_MESH = plsc.VectorSubcoreMesh(core_axis_name="c", subcore_axis_name="s")


def _conv_scratch(be, bpr):
    return [
        pltpu.VMEM((bpr, be), i32),
        pltpu.VMEM((bpr, be), i32),
        pltpu.VMEM((be,), i32),
        pltpu.VMEM((be,), i32),
        pltpu.VMEM((be,), i32),
        pltpu.VMEM((be,), i32),
        pltpu.VMEM((be, 16), f32),
        pltpu.VMEM((be, 16), f32),
        pltpu.VMEM((be, 16), f32),
        pltpu.VMEM((be, 16), f32),
        pltpu.VMEM((be, 16), f32),
        pltpu.VMEM((be, 16), f32),
        pltpu.VMEM((be, 16), f32),
        pltpu.VMEM((be, 16), f32),
        pltpu.VMEM((be * 16,), f32),
        pltpu.VMEM((be, 128), f32),
        pltpu.VMEM((be, 128), f32),
    ]


_SC_PARAMS = pltpu.CompilerParams(needs_layout_passes=False,
                                  use_tc_tiling_on_sc=False)

_sc1_call = pl.kernel(
    _sc1_body,
    out_type=(jax.ShapeDtypeStruct((EP1,), f32),
              jax.ShapeDtypeStruct((2, NP, 128), f32)),
    mesh=_SC_MESH,
    compiler_params=_SC_PARAMS,
    scratch_types=_conv_scratch(BE1, BPR1) + [
        pltpu.VMEM((BE1,), f32),
        pltpu.VMEM_SHARED((NP, 128), f32),
        pltpu.VMEM_SHARED((NP, 16), f32),
        pltpu.SemaphoreType.DMA,
        pltpu.SemaphoreType.DMA,
        pltpu.SemaphoreType.DMA,
        pltpu.SemaphoreType.DMA,
    ],
)

_sc2_call = pl.kernel(
    _sc2_body,
    out_type=jax.ShapeDtypeStruct((2, NP, 128), f32),
    mesh=_SC_MESH,
    compiler_params=_SC_PARAMS,
    scratch_types=_conv_scratch(BE2, BPR2) + [
        pltpu.VMEM((CPT2,), f32),
        pltpu.VMEM((16,), i32),
        pltpu.VMEM((16, 16), i32),
        pltpu.VMEM_SHARED((NP, 128), f32),
        pltpu.VMEM_SHARED((NP, 16), f32),
        pltpu.VMEM_SHARED((16, 16), i32),
        pltpu.VMEM_SHARED((16, 16), i32),
        pltpu.SemaphoreType.DMA,
        pltpu.SemaphoreType.DMA,
        pltpu.SemaphoreType.DMA,
        pltpu.SemaphoreType.DMA,
    ],
)


def _att_mat(att_s, att_d):
    fl_s = att_s.reshape(HID)
    fl_d = att_d.reshape(HID)
    fidx = jnp.arange(HID)
    head = fidx // HEAD_DIM
    m = jnp.zeros((HID, 32), f32)
    m = m.at[fidx, head].set(fl_s)
    m = m.at[fidx, 16 + head].set(fl_d)
    return m


def kernel(x, edge_index, res_W, res_b, c1_W, c1_as, c1_ad, c1_b,
           bn1_g, bn1_b, bn1_m, bn1_v, c2_W, c2_as, c2_ad, c2_b,
           bn2_g, bn2_b, bn2_m, bn2_v, f1_W, f1_b,
           fbn_g, fbn_b, fbn_m, fbn_v, f2_W, f2_b):
    xpad = jnp.zeros((NP, IN_SIZE), f32).at[:N].set(x)

    loops = jnp.arange(N, dtype=i32)
    src0 = edge_index[0].astype(i32)
    dst0 = edge_index[1].astype(i32)
    padN = jnp.full((EP1 - NE1,), N, i32)
    src1 = jnp.concatenate([src0, loops, padN])
    dst1 = jnp.concatenate([dst0, loops, padN])
    padN2 = jnp.full((EP2 - NE2,), N, i32)
    src2 = jnp.concatenate([src0, loops, loops, padN2])
    dst2 = jnp.concatenate([dst0, loops, loops, padN2])

    am1 = _att_mat(c1_as, c1_ad)
    am2 = _att_mat(c2_as, c2_ad)

    xp, xl1, as1, ad1 = pl.pallas_call(
        _tc1_body,
        grid=(NBLK,),
        in_specs=[_row_spec(IN_SIZE), _full_spec((IN_SIZE, HID)),
                  _full_spec((1, HID)), _full_spec((HID, HID)),
                  _full_spec((HID, 32))],
        out_specs=[_row_spec(HID),
                   pl.BlockSpec((2, RB, 128), lambda i: (0, i, 0)),
                   _row_spec(16), _row_spec(16)],
        out_shape=[jax.ShapeDtypeStruct((NP, HID), f32),
                   jax.ShapeDtypeStruct((2, NP, 128), f32),
                   jax.ShapeDtypeStruct((NP, 16), f32),
                   jax.ShapeDtypeStruct((NP, 16), f32)],
    )(xpad, res_W, res_b.reshape(1, HID), c1_W, am1)

    scores1, msgs1 = _sc1_call(src1.reshape(EP1 // BE1, BE1),
                               dst1.reshape(EP1 // BE1, BE1),
                               as1, ad1, xl1)

    pv1 = jnp.stack([c1_b, bn1_g, bn1_b, bn1_m, bn1_v])
    xl2, as2, ad2 = pl.pallas_call(
        _tc2_body,
        grid=(NBLK,),
        in_specs=[pl.BlockSpec((2, RB, 128), lambda i: (0, i, 0)),
                  _full_spec((5, HID)), _full_spec((HID, HID)),
                  _full_spec((HID, 32))],
        out_specs=[pl.BlockSpec((2, RB, 128), lambda i: (0, i, 0)),
                   _row_spec(16), _row_spec(16)],
        out_shape=[jax.ShapeDtypeStruct((2, NP, 128), f32),
                   jax.ShapeDtypeStruct((NP, 16), f32),
                   jax.ShapeDtypeStruct((NP, 16), f32)],
    )(msgs1, pv1, c2_W, am2)

    scores2 = jnp.concatenate([
        scores1[:NE1],
        jnp.full((N,), 2.0, f32),
        jnp.full((EP2 - NE2,), -1.0, f32)
    ])
    msgs2 = _sc2_call(src2.reshape(EP2 // BE2, BE2),
                      dst2.reshape(EP2 // BE2, BE2),
                      as2, ad2, xl2, scores2)

    pv2 = jnp.stack([c2_b, bn2_g, bn2_b, bn2_m, bn2_v])
    f2w_pad = jnp.zeros((HID // 2, 128), f32).at[:, :OUT_SIZE].set(f2_W)
    qv = jnp.stack([f1_b, fbn_g, fbn_b, fbn_m, fbn_v,
                    jnp.zeros((HID // 2,), f32)])
    qv = jnp.zeros((6, 128), f32).at[:, :HID // 2].set(qv)
    qv = qv.at[5, :OUT_SIZE].set(f2_b)
    out = pl.pallas_call(
        _tc3_body,
        grid=(NBLK,),
        in_specs=[pl.BlockSpec((2, RB, 128), lambda i: (0, i, 0)),
                  _full_spec((5, HID)), _row_spec(HID),
                  _full_spec((HID, HID // 2)), _full_spec((6, 128)),
                  _full_spec((HID // 2, 128))],
        out_specs=[_row_spec(128)],
        out_shape=[jax.ShapeDtypeStruct((NP, 128), f32)],
    )(msgs2, pv2, xp, f1_W, qv, f2w_pad)[0]

    return out[:N, :OUT_SIZE]


# speedup vs baseline: 14.1407x; 1.0301x over previous
import jax
import jax.numpy as jnp
from jax import lax
from jax.experimental import pallas as pl
from jax.experimental.pallas import tpu as pltpu
from jax.experimental.pallas import tpu_sc as plsc

N = 10000
IN_SIZE = 128
HEADS = 8
HEAD_DIM = 32
HID = 256
OUT_SIZE = 40
E = 320000

NP = 10112
RB = 128
NBLK = NP // RB

NE1 = E + N
EP1 = 331776
NE2 = NE1 + N
EP2 = 344064
K_TOP = int(NE1 * 0.3)
KSEL = K_TOP + N

NSUB = 16
CPT1 = EP1 // NSUB
CPT2 = EP2 // NSUB
RPT = NP // NSUB

BE1 = 64
BPR1 = 12
NRND1 = CPT1 // (BE1 * BPR1)
BE2 = 32
BPR2 = 12
NRND2 = CPT2 // (BE2 * BPR2)

f32 = jnp.float32
i32 = jnp.int32


def _bn(x, g, b, m, v):
    return (x - m) * jax.lax.rsqrt(v + 1e-5) * g + b


def _elu(x):
    return jnp.where(x > 0, x, jnp.exp(jnp.minimum(x, 0.0)) - 1.0)


def _tc1_body(x_ref, rw_ref, rb_ref, c1w_ref, am_ref,
              xp_ref, xl_ref, as_ref, ad_ref):
    xp = jnp.dot(x_ref[...], rw_ref[...], preferred_element_type=f32)
    xp = xp + rb_ref[...]
    xp_ref[...] = xp
    xl = jnp.dot(xp, c1w_ref[...], preferred_element_type=f32)
    xl_ref[0] = xl[:, :128]
    xl_ref[1] = xl[:, 128:]
    a = jnp.dot(xl, am_ref[...], preferred_element_type=f32)
    as_ref[...] = a[:, :16]
    ad_ref[...] = a[:, 16:32]


def _tc2_body(m_ref, pv_ref, c2w_ref, am_ref,
              xl_ref, as_ref, ad_ref):
    msg = jnp.concatenate([m_ref[0], m_ref[1]], axis=1)
    pv = pv_ref[...]
    h1 = msg + pv[0:1, :]
    h = _elu(_bn(h1, pv[1:2, :], pv[2:3, :], pv[3:4, :], pv[4:5, :]))
    xl = jnp.dot(h, c2w_ref[...], preferred_element_type=f32)
    xl_ref[0] = xl[:, :128]
    xl_ref[1] = xl[:, 128:]
    a = jnp.dot(xl, am_ref[...], preferred_element_type=f32)
    as_ref[...] = a[:, :16]
    ad_ref[...] = a[:, 16:32]


def _tc3_body(m_ref, pv_ref, xp_ref, f1w_ref, q_ref, f2w_ref,
              out_ref):
    msg = jnp.concatenate([m_ref[0], m_ref[1]], axis=1)
    pv = pv_ref[...]
    h1 = msg + pv[0:1, :]
    h2 = _elu(_bn(h1, pv[1:2, :], pv[2:3, :], pv[3:4, :], pv[4:5, :]))
    h2 = h2 + xp_ref[...]
    z = jnp.dot(h2, f1w_ref[...], preferred_element_type=f32)
    q = q_ref[...]
    z = z + q[0:1, :]
    z = jnp.maximum(_bn(z, q[1:2, :], q[2:3, :], q[3:4, :], q[4:5, :]), 0.0)
    lg = jnp.dot(z, f2w_ref[...], preferred_element_type=f32)
    lg = lg + q[5:6, :]
    col = lax.broadcasted_iota(i32, lg.shape, 1)
    lgm = jnp.where(col < OUT_SIZE, lg, -1e30)
    mx = jnp.max(lgm, axis=1, keepdims=True)
    ex = jnp.where(col < OUT_SIZE, jnp.exp(lg - mx), 0.0)
    s = jnp.sum(ex, axis=1, keepdims=True)
    out_ref[...] = lg - mx - jnp.log(s)


def _row_spec(w):
    return pl.BlockSpec((RB, w), lambda i: (i, 0))


def _full_spec(shape):
    nd = len(shape)
    return pl.BlockSpec(shape, lambda i: (0,) * nd)


def _zero_vmem(ref, rows, width):
    z = jnp.zeros((16,), f32)

    def body(j, _):
        for v in range(width // 16):
            ref[j, pl.ds(16 * v, 16)] = z
        return 0

    lax.fori_loop(0, rows, body, 0)


def _zero_shared_stripe(zb_ref, chunk, sh_ref, row0, rows):
    nfull = rows // chunk
    rem = rows - nfull * chunk
    for i in range(nfull):
        pltpu.sync_copy(zb_ref, sh_ref.at[pl.ds(row0 + i * chunk, chunk)])
    if rem:
        pltpu.sync_copy(zb_ref.at[pl.ds(0, rem)],
                        sh_ref.at[pl.ds(row0 + nfull * chunk, rem)])


def _edge_alpha_num(ea_s_ref, ea_d_ref, j, mask8):
    va = ea_s_ref[j] + ea_d_ref[j]
    va = jnp.where(va >= 0.0, va, 0.2 * va)
    ve = jnp.exp(va)
    return jnp.where(mask8, ve, 0.0)


def _pipelined_blocks(bpr, fire, compute, sfire):
    handles = [None, None]
    shandles = [None, None]
    handles[0] = fire(0, 0)
    for b in range(bpr):
        cur = b % 2
        nxt = 1 - cur
        if b + 1 < bpr:
            if shandles[nxt] is not None:
                shandles[nxt].wait()
            handles[nxt] = fire(b + 1, nxt)
        for d in handles[cur]:
            d.wait()
        compute(b, cur)
        shandles[cur] = sfire(b, cur)
    for i in (0, 1):
        if shandles[i] is not None:
            shandles[i].wait()


def _sc_conv(be, bpr, nrnd, src2d_h, dst2d_h, asr_h, adr_h, xl_h,
             selm_src, write_scores, scores_out, sums_hbm, row0,
             sslab, dslab, sidx, didx, ea_s, ea_d, exb, esum, alf, msg, scr,
             sums_sh, acc_sh, semg, sems, c, s, mask8, lane):
    rbase0 = s * (bpr * nrnd)
    ebase0 = s * (be * bpr * nrnd)

    def load_slabs(r):
        row = pl.multiple_of(rbase0 + r * bpr, 4)
        pltpu.sync_copy(src2d_h.at[pl.ds(row, bpr)], sslab)
        pltpu.sync_copy(dst2d_h.at[pl.ds(row, bpr)], dslab)

    def eb_of(r, b):
        return pl.multiple_of(ebase0 + (r * bpr + b) * be, be)

    def p1_round(r, _):
        load_slabs(r)

        def fire(b, i):
            for v in range(be // 16):
                sidx[i][pl.ds(16 * v, 16)] = sslab[b, pl.ds(16 * v, 16)]
                didx[i][pl.ds(16 * v, 16)] = dslab[b, pl.ds(16 * v, 16)]
            return [
                pltpu.async_copy(asr_h.at[sidx[i]], ea_s[i], semg[i]),
                pltpu.async_copy(adr_h.at[didx[i]], ea_d[i], semg[i]),
            ]

        def compute(b, i):
            loc0 = (r * bpr + b) * be

            def rows(j, _):
                ve = _edge_alpha_num(ea_s[i], ea_d[i], j, mask8)
                if selm_src is not None:
                    ve = ve * selm_src(loc0 + j)
                exb[i][j] = ve
                return 0

            lax.fori_loop(0, be, rows, 0)

        def sfire(b, i):
            return pltpu.async_copy(exb[i], sums_sh.at[didx[i]], sems[i],
                                    add=True)

        _pipelined_blocks(bpr, fire, compute, sfire)
        return 0

    lax.fori_loop(0, nrnd, p1_round, 0)
    plsc.subcore_barrier()

    sums_half = sums_hbm.at[c]
    nfull = RPT // 128
    for i in range(nfull):
        pltpu.sync_copy(sums_sh.at[pl.ds(row0 + i * 128, 128)],
                        sums_half.at[pl.ds(row0 + i * 128, 128)])
    rem = RPT - nfull * 128
    if rem:
        pltpu.sync_copy(sums_sh.at[pl.ds(row0 + RPT - rem, rem)],
                        sums_half.at[pl.ds(row0 + RPT - rem, rem)])
    plsc.subcore_barrier()

    xl_src = xl_h.at[c]
    sums_src = sums_hbm.at[c]

    def p2_round(r, _):
        load_slabs(r)

        def fire(b, i):
            for v in range(be // 16):
                sidx[i][pl.ds(16 * v, 16)] = sslab[b, pl.ds(16 * v, 16)]
                didx[i][pl.ds(16 * v, 16)] = dslab[b, pl.ds(16 * v, 16)]
            return [
                pltpu.async_copy(asr_h.at[sidx[i]], ea_s[i], semg[i]),
                pltpu.async_copy(adr_h.at[didx[i]], ea_d[i], semg[i]),
                pltpu.async_copy(sums_src.at[didx[i]], esum[i], semg[i]),
                pltpu.async_copy(xl_src.at[sidx[i]], msg[i], semg[i]),
            ]

        def compute(b, i):
            loc0 = (r * bpr + b) * be

            def rows(j, _):
                ve = _edge_alpha_num(ea_s[i], ea_d[i], j, mask8)
                if selm_src is not None:
                    ve = ve * selm_src(loc0 + j)
                al = ve / (esum[i][j] + 1e-16)
                alf[pl.ds(pl.multiple_of(j * 16, 16), 16)] = al
                jv = jnp.full((16,), j * 16, i32)
                for v in range(8):
                    hv = c * 4 + (v // 2)
                    am = plsc.load_gather(alf, [jv + hv])
                    sl = pl.ds(16 * v, 16)
                    msg[i][j, sl] = msg[i][j, sl] * am
                return 0

            lax.fori_loop(0, be, rows, 0)

            if write_scores:
                @pl.when(c == 0)
                def _():
                    eb = eb_of(r, b)
                    for j16 in range(be // 16):
                        rv = (jnp.full((16,), j16 * 16, i32) + lane) * 16
                        acc = jnp.zeros((16,), f32)
                        for h in range(8):
                            acc = acc + plsc.load_gather(alf, [rv + h])
                        sc = acc * 0.125
                        eid = jnp.full((16,), eb + j16 * 16, i32) + lane
                        sc = jnp.where(eid < NE1, sc, -1.0)
                        scr[pl.ds(j16 * 16, 16)] = sc
                    pltpu.sync_copy(scr, scores_out.at[pl.ds(eb, be)])

        def sfire(b, i):
            return pltpu.async_copy(msg[i], acc_sh.at[didx[i]], sems[i],
                                    add=True)

        _pipelined_blocks(bpr, fire, compute, sfire)
        return 0

    lax.fori_loop(0, nrnd, p2_round, 0)
    plsc.subcore_barrier()


def _write_out_half(msgs_out, acc_sh, c, row0, chunk):
    out_half = msgs_out.at[c]
    nfull = RPT // chunk
    rem = RPT - nfull * chunk
    for i in range(nfull):
        pltpu.sync_copy(acc_sh.at[pl.ds(row0 + i * chunk, chunk)],
                        out_half.at[pl.ds(row0 + i * chunk, chunk)])
    if rem:
        pltpu.sync_copy(acc_sh.at[pl.ds(row0 + RPT - rem, rem)],
                        out_half.at[pl.ds(row0 + RPT - rem, rem)])


def _sc1_body(src2d_h, dst2d_h, asr_h, adr_h, xl_h,
              scores_out, msgs_out, sums_out,
              sslab, dslab, sidxA, sidxB, didxA, didxB,
              ea_sA, ea_sB, ea_dA, ea_dB,
              exbA, exbB, esumA, esumB, alf, msgA, msgB, scr,
              acc_sh, sums_sh, semgA, semgB, semsA, semsB):
    c = lax.axis_index("c")
    s = lax.axis_index("s")
    lane = lax.iota(i32, 16)
    mask8 = lane < 8
    row0 = pl.multiple_of(s * RPT, 8)

    _zero_vmem(msgA, BE1, 128)
    _zero_shared_stripe(msgA, BE1, acc_sh, row0, RPT)
    _zero_vmem(exbA, BE1, 16)
    _zero_shared_stripe(exbA, BE1, sums_sh, row0, RPT)
    plsc.subcore_barrier()

    _sc_conv(BE1, BPR1, NRND1, src2d_h, dst2d_h, asr_h, adr_h,
             xl_h, None, True, scores_out, sums_out, row0,
             sslab, dslab, [sidxA, sidxB], [didxA, didxB],
             [ea_sA, ea_sB], [ea_dA, ea_dB],
             [exbA, exbB], [esumA, esumB], alf, [msgA, msgB], scr,
             sums_sh, acc_sh, [semgA, semgB], [semsA, semsB],
             c, s, mask8, lane)

    _write_out_half(msgs_out, acc_sh, c, row0, BE1)


def _sc2_body(src2d_h, dst2d_h, asr_h, adr_h, xl_h, scores2_h,
              msgs_out, sums_out,
              sslab, dslab, sidxA, sidxB, didxA, didxB,
              ea_sA, ea_sB, ea_dA, ea_dB,
              exbA, exbB, esumA, esumB, alf, msgA, msgB,
              keys, cb, stb,
              acc_sh, sums_sh, stage_sh, stage2_sh,
              semgA, semgB, semsA, semsB):
    c = lax.axis_index("c")
    s = lax.axis_index("s")
    lane = lax.iota(i32, 16)
    mask8 = lane < 8
    row0 = pl.multiple_of(s * RPT, 8)
    base0 = pl.multiple_of(s * CPT2, 8)

    _zero_vmem(msgA, BE2, 128)
    _zero_shared_stripe(msgA, BE2, acc_sh, row0, RPT)
    _zero_vmem(exbA, BE2, 16)
    _zero_shared_stripe(exbA, BE2, sums_sh, row0, RPT)
    pltpu.sync_copy(scores2_h.at[pl.ds(base0, CPT2)], keys)
    plsc.subcore_barrier()

    nv = CPT2 // 16

    def count_gt(thr):
        def cbody(v, acc):
            kf = keys[pl.ds(16 * v, 16)]
            ki = plsc.bitcast(kf, i32)
            return acc + jnp.where(ki > thr, 1, 0).astype(i32)
        acc = lax.fori_loop(0, nv, cbody, jnp.zeros((16,), i32))
        return jnp.sum(acc)

    def stage_scalar(val, sh):
        cb[...] = jnp.full((16,), val, i32)
        pltpu.sync_copy(cb, sh.at[s])

    def read_total(sh):
        pltpu.sync_copy(sh, stb)
        tot = jnp.zeros((16,), i32)
        for r in range(16):
            tot = tot + stb[r]
        return tot[0]

    def wcond(carry):
        lo, hi = carry
        return hi - lo > 1

    def wbody(carry):
        lo, hi = carry
        mid = (lo + hi) // 2
        cnt = count_gt(mid)
        stage_scalar(cnt, stage_sh)
        plsc.subcore_barrier()
        tot = read_total(stage_sh)
        plsc.subcore_barrier()
        pred = tot >= KSEL
        return (jnp.where(pred, mid, lo), jnp.where(pred, hi, mid))

    lo0 = jnp.asarray(-2, i32)
    hi0 = jnp.asarray(1 << 30, i32)
    _, thr = lax.while_loop(wcond, wbody, (lo0, hi0))

    def gq_body(v, acc):
        g, q = acc
        kf = keys[pl.ds(16 * v, 16)]
        ki = plsc.bitcast(kf, i32)
        g = g + jnp.where(ki > thr, 1, 0).astype(i32)
        q = q + jnp.where(ki == thr, 1, 0).astype(i32)
        return (g, q)

    gv, qv = lax.fori_loop(0, nv, gq_body,
                           (jnp.zeros((16,), i32), jnp.zeros((16,), i32)))
    stage_scalar(jnp.sum(gv), stage_sh)
    stage_scalar(jnp.sum(qv), stage2_sh)
    plsc.subcore_barrier()
    g_tot = read_total(stage_sh)
    pltpu.sync_copy(stage2_sh, stb)
    pref = jnp.asarray(0, i32)
    for r in range(16):
        qr = stb[r][0]
        pref = pref + jnp.where(r < s, qr, 0)
    plsc.subcore_barrier()
    need = KSEL - g_tot

    def sel_body(v, cnt):
        kf = keys[pl.ds(16 * v, 16)]
        ki = plsc.bitcast(kf, i32)
        gt = ki > thr
        eq = ki == thr
        eqi = jnp.where(eq, 1, 0).astype(i32)
        pre = plsc.cumsum(eqi)
        rank = jnp.full((16,), cnt, i32) + pre - 1
        keep = jnp.logical_or(gt, jnp.logical_and(eq, rank < need))
        keys[pl.ds(16 * v, 16)] = jnp.where(keep, 1.0, 0.0)
        return cnt + jnp.sum(eqi)

    lax.fori_loop(0, nv, sel_body, pref)
    plsc.subcore_barrier()

    def selv(loc):
        return plsc.load_gather(keys, [jnp.full((16,), loc, i32)])

    _sc_conv(BE2, BPR2, NRND2, src2d_h, dst2d_h, asr_h, adr_h,
             xl_h, selv, False, None, sums_out, row0,
             sslab, dslab, [sidxA, sidxB], [didxA, didxB],
             [ea_sA, ea_sB], [ea_dA, ea_dB],
             [exbA, exbB], [esumA, esumB], alf, [msgA, msgB], None,
             sums_sh, acc_sh, [semgA, semgB], [semsA, semsB],
             c, s, mask8, lane)

    _write_out_half(msgs_out, acc_sh, c, row0, BE2)


_SC_MESH = plsc.VectorSubcoreMesh(core_axis_name="c", subcore_axis_name="s")


def _conv_scratch(be, bpr):
    return [
        pltpu.VMEM((bpr, be), i32),
        pltpu.VMEM((bpr, be), i32),
        pltpu.VMEM((be,), i32),
        pltpu.VMEM((be,), i32),
        pltpu.VMEM((be,), i32),
        pltpu.VMEM((be,), i32),
        pltpu.VMEM((be, 16), f32),
        pltpu.VMEM((be, 16), f32),
        pltpu.VMEM((be, 16), f32),
        pltpu.VMEM((be, 16), f32),
        pltpu.VMEM((be, 16), f32),
        pltpu.VMEM((be, 16), f32),
        pltpu.VMEM((be, 16), f32),
        pltpu.VMEM((be, 16), f32),
        pltpu.VMEM((be * 16,), f32),
        pltpu.VMEM((be, 128), f32),
        pltpu.VMEM((be, 128), f32),
    ]


_SC_PARAMS = pltpu.CompilerParams(needs_layout_passes=False,
                                  use_tc_tiling_on_sc=False)

_sc1_call = pl.kernel(
    _sc1_body,
    out_type=(jax.ShapeDtypeStruct((EP1,), f32),
              jax.ShapeDtypeStruct((2, NP, 128), f32),
              jax.ShapeDtypeStruct((2, NP, 16), f32)),
    mesh=_SC_MESH,
    compiler_params=_SC_PARAMS,
    scratch_types=_conv_scratch(BE1, BPR1) + [
        pltpu.VMEM((BE1,), f32),
        pltpu.VMEM_SHARED((NP, 128), f32),
        pltpu.VMEM_SHARED((NP, 16), f32),
        pltpu.SemaphoreType.DMA,
        pltpu.SemaphoreType.DMA,
        pltpu.SemaphoreType.DMA,
        pltpu.SemaphoreType.DMA,
    ],
)

_sc2_call = pl.kernel(
    _sc2_body,
    out_type=(jax.ShapeDtypeStruct((2, NP, 128), f32),
              jax.ShapeDtypeStruct((2, NP, 16), f32)),
    mesh=_SC_MESH,
    compiler_params=_SC_PARAMS,
    scratch_types=_conv_scratch(BE2, BPR2) + [
        pltpu.VMEM((CPT2,), f32),
        pltpu.VMEM((16,), i32),
        pltpu.VMEM((16, 16), i32),
        pltpu.VMEM_SHARED((NP, 128), f32),
        pltpu.VMEM_SHARED((NP, 16), f32),
        pltpu.VMEM_SHARED((16, 16), i32),
        pltpu.VMEM_SHARED((16, 16), i32),
        pltpu.SemaphoreType.DMA,
        pltpu.SemaphoreType.DMA,
        pltpu.SemaphoreType.DMA,
        pltpu.SemaphoreType.DMA,
    ],
)


def _att_mat(att_s, att_d):
    fl_s = att_s.reshape(HID)
    fl_d = att_d.reshape(HID)
    fidx = jnp.arange(HID)
    head = fidx // HEAD_DIM
    m = jnp.zeros((HID, 32), f32)
    m = m.at[fidx, head].set(fl_s)
    m = m.at[fidx, 16 + head].set(fl_d)
    return m


def kernel(x, edge_index, res_W, res_b, c1_W, c1_as, c1_ad, c1_b,
           bn1_g, bn1_b, bn1_m, bn1_v, c2_W, c2_as, c2_ad, c2_b,
           bn2_g, bn2_b, bn2_m, bn2_v, f1_W, f1_b,
           fbn_g, fbn_b, fbn_m, fbn_v, f2_W, f2_b):
    xpad = jnp.zeros((NP, IN_SIZE), f32).at[:N].set(x)

    loops = jnp.arange(N, dtype=i32)
    src0 = edge_index[0].astype(i32)
    dst0 = edge_index[1].astype(i32)
    padN = jnp.full((EP1 - NE1,), N, i32)
    src1 = jnp.concatenate([src0, loops, padN])
    dst1 = jnp.concatenate([dst0, loops, padN])
    padN2 = jnp.full((EP2 - NE2,), N, i32)
    src2 = jnp.concatenate([src0, loops, loops, padN2])
    dst2 = jnp.concatenate([dst0, loops, loops, padN2])

    am1 = _att_mat(c1_as, c1_ad)
    am2 = _att_mat(c2_as, c2_ad)

    xp, xl1, as1, ad1 = pl.pallas_call(
        _tc1_body,
        grid=(NBLK,),
        in_specs=[_row_spec(IN_SIZE), _full_spec((IN_SIZE, HID)),
                  _full_spec((1, HID)), _full_spec((HID, HID)),
                  _full_spec((HID, 32))],
        out_specs=[_row_spec(HID),
                   pl.BlockSpec((2, RB, 128), lambda i: (0, i, 0)),
                   _row_spec(16), _row_spec(16)],
        out_shape=[jax.ShapeDtypeStruct((NP, HID), f32),
                   jax.ShapeDtypeStruct((2, NP, 128), f32),
                   jax.ShapeDtypeStruct((NP, 16), f32),
                   jax.ShapeDtypeStruct((NP, 16), f32)],
    )(xpad, res_W, res_b.reshape(1, HID), c1_W, am1)

    scores1, msgs1, _ = _sc1_call(src1.reshape(EP1 // BE1, BE1),
                                  dst1.reshape(EP1 // BE1, BE1),
                                  as1, ad1, xl1)

    pv1 = jnp.stack([c1_b, bn1_g, bn1_b, bn1_m, bn1_v])
    xl2, as2, ad2 = pl.pallas_call(
        _tc2_body,
        grid=(NBLK,),
        in_specs=[pl.BlockSpec((2, RB, 128), lambda i: (0, i, 0)),
                  _full_spec((5, HID)), _full_spec((HID, HID)),
                  _full_spec((HID, 32))],
        out_specs=[pl.BlockSpec((2, RB, 128), lambda i: (0, i, 0)),
                   _row_spec(16), _row_spec(16)],
        out_shape=[jax.ShapeDtypeStruct((2, NP, 128), f32),
                   jax.ShapeDtypeStruct((NP, 16), f32),
                   jax.ShapeDtypeStruct((NP, 16), f32)],
    )(msgs1, pv1, c2_W, am2)

    scores2 = jnp.concatenate([
        scores1[:NE1],
        jnp.full((N,), 2.0, f32),
        jnp.full((EP2 - NE2,), -1.0, f32)
    ])
    msgs2, _ = _sc2_call(src2.reshape(EP2 // BE2, BE2),
                         dst2.reshape(EP2 // BE2, BE2),
                         as2, ad2, xl2, scores2)

    pv2 = jnp.stack([c2_b, bn2_g, bn2_b, bn2_m, bn2_v])
    f2w_pad = jnp.zeros((HID // 2, 128), f32).at[:, :OUT_SIZE].set(f2_W)
    qv = jnp.stack([f1_b, fbn_g, fbn_b, fbn_m, fbn_v,
                    jnp.zeros((HID // 2,), f32)])
    qv = jnp.zeros((6, 128), f32).at[:, :HID // 2].set(qv)
    qv = qv.at[5, :OUT_SIZE].set(f2_b)
    out = pl.pallas_call(
        _tc3_body,
        grid=(NBLK,),
        in_specs=[pl.BlockSpec((2, RB, 128), lambda i: (0, i, 0)),
                  _full_spec((5, HID)), _row_spec(HID),
                  _full_spec((HID, HID // 2)), _full_spec((6, 128)),
                  _full_spec((HID // 2, 128))],
        out_specs=[_row_spec(128)],
        out_shape=[jax.ShapeDtypeStruct((NP, 128), f32)],
    )(msgs2, pv2, xp, f1_W, qv, f2w_pad)[0]

    return out[:N, :OUT_SIZE]


# speedup vs baseline: 17.0629x; 1.2066x over previous
import jax
import jax.numpy as jnp
from jax import lax
from jax.experimental import pallas as pl
from jax.experimental.pallas import tpu as pltpu
from jax.experimental.pallas import tpu_sc as plsc

N = 10000
IN_SIZE = 128
HEADS = 8
HEAD_DIM = 32
HID = 256
OUT_SIZE = 40
E = 320000

NP = 10112
RB = 128
NBLK = NP // RB

NE1 = E + N
EP1 = 331776
NE2 = NE1 + N
EP2 = 344064
K_TOP = int(NE1 * 0.3)
KSEL = K_TOP + N

NSUB = 16
CPT1 = EP1 // NSUB
CPT2 = EP2 // NSUB
RPT = NP // NSUB

BE1 = 64
BPR1 = 12
NRND1 = CPT1 // (BE1 * BPR1)
BE2 = 32
BPR2 = 12
NRND2 = CPT2 // (BE2 * BPR2)

f32 = jnp.float32
i32 = jnp.int32


def _bn(x, g, b, m, v):
    return (x - m) * jax.lax.rsqrt(v + 1e-5) * g + b


def _elu(x):
    return jnp.where(x > 0, x, jnp.exp(jnp.minimum(x, 0.0)) - 1.0)


def _tc1_body(x_ref, rw_ref, rb_ref, c1w_ref, am_ref,
              xp_ref, xl_ref, as_ref, ad_ref):
    xp = jnp.dot(x_ref[...], rw_ref[...], preferred_element_type=f32)
    xp = xp + rb_ref[...]
    xp_ref[...] = xp
    xl = jnp.dot(xp, c1w_ref[...], preferred_element_type=f32)
    xl_ref[0] = xl[:, :128]
    xl_ref[1] = xl[:, 128:]
    a = jnp.dot(xl, am_ref[...], preferred_element_type=f32)
    as_ref[...] = a[:, :16]
    ad_ref[...] = a[:, 16:32]


def _tc2_body(m_ref, pv_ref, c2w_ref, am_ref,
              xl_ref, as_ref, ad_ref):
    msg = jnp.concatenate([m_ref[0], m_ref[1]], axis=1)
    pv = pv_ref[...]
    h1 = msg + pv[0:1, :]
    h = _elu(_bn(h1, pv[1:2, :], pv[2:3, :], pv[3:4, :], pv[4:5, :]))
    xl = jnp.dot(h, c2w_ref[...], preferred_element_type=f32)
    xl_ref[0] = xl[:, :128]
    xl_ref[1] = xl[:, 128:]
    a = jnp.dot(xl, am_ref[...], preferred_element_type=f32)
    as_ref[...] = a[:, :16]
    ad_ref[...] = a[:, 16:32]


def _tc3_body(m_ref, pv_ref, xp_ref, f1w_ref, q_ref, f2w_ref,
              out_ref):
    msg = jnp.concatenate([m_ref[0], m_ref[1]], axis=1)
    pv = pv_ref[...]
    h1 = msg + pv[0:1, :]
    h2 = _elu(_bn(h1, pv[1:2, :], pv[2:3, :], pv[3:4, :], pv[4:5, :]))
    h2 = h2 + xp_ref[...]
    z = jnp.dot(h2, f1w_ref[...], preferred_element_type=f32)
    q = q_ref[...]
    z = z + q[0:1, :]
    z = jnp.maximum(_bn(z, q[1:2, :], q[2:3, :], q[3:4, :], q[4:5, :]), 0.0)
    lg = jnp.dot(z, f2w_ref[...], preferred_element_type=f32)
    lg = lg + q[5:6, :]
    col = lax.broadcasted_iota(i32, lg.shape, 1)
    lgm = jnp.where(col < OUT_SIZE, lg, -1e30)
    mx = jnp.max(lgm, axis=1, keepdims=True)
    ex = jnp.where(col < OUT_SIZE, jnp.exp(lg - mx), 0.0)
    s = jnp.sum(ex, axis=1, keepdims=True)
    out_ref[...] = lg - mx - jnp.log(s)


def _row_spec(w):
    return pl.BlockSpec((RB, w), lambda i: (i, 0))


def _full_spec(shape):
    nd = len(shape)
    return pl.BlockSpec(shape, lambda i: (0,) * nd)


def _zero_vmem(ref, rows, width):
    z = jnp.zeros((16,), f32)

    def body(j, _):
        for v in range(width // 16):
            ref[j, pl.ds(16 * v, 16)] = z
        return 0

    lax.fori_loop(0, rows, body, 0)


def _zero_shared_stripe(zb_ref, chunk, sh_ref, row0, rows):
    nfull = rows // chunk
    rem = rows - nfull * chunk
    for i in range(nfull):
        pltpu.sync_copy(zb_ref, sh_ref.at[pl.ds(row0 + i * chunk, chunk)])
    if rem:
        pltpu.sync_copy(zb_ref.at[pl.ds(0, rem)],
                        sh_ref.at[pl.ds(row0 + nfull * chunk, rem)])


def _edge_alpha_num(ea_s_ref, ea_d_ref, j, mask8):
    va = ea_s_ref[j] + ea_d_ref[j]
    va = jnp.where(va >= 0.0, va, 0.2 * va)
    ve = jnp.exp(va)
    return jnp.where(mask8, ve, 0.0)


def _pipelined_blocks(bpr, fire, compute, sfire):
    handles = [None, None]
    shandles = [None, None]
    handles[0] = fire(0, 0)
    for b in range(bpr):
        cur = b % 2
        nxt = 1 - cur
        if b + 1 < bpr:
            if shandles[nxt] is not None:
                for d in shandles[nxt]:
                    d.wait()
            handles[nxt] = fire(b + 1, nxt)
        for d in handles[cur]:
            d.wait()
        compute(b, cur)
        shandles[cur] = sfire(b, cur)
    for i in (0, 1):
        if shandles[i] is not None:
            for d in shandles[i]:
                d.wait()


def _sc_conv(be, bpr, nrnd, src2d_h, dst2d_h, asr_h, adr_h, xl_h,
             selm_src, write_scores, scores_out, sums_hbm, row0,
             sslab, dslab, sidx, didx, ea_s, ea_d, exb, esum, alf, msg, scr,
             sums_sh, acc_sh, semg, sems, c, s, mask8, lane):
    rbase0 = s * (bpr * nrnd)
    ebase0 = s * (be * bpr * nrnd)

    def load_slabs(r):
        row = pl.multiple_of(rbase0 + r * bpr, 4)
        pltpu.sync_copy(src2d_h.at[pl.ds(row, bpr)], sslab)
        pltpu.sync_copy(dst2d_h.at[pl.ds(row, bpr)], dslab)

    def eb_of(r, b):
        return pl.multiple_of(ebase0 + (r * bpr + b) * be, be)

    def p1_round(r, _):
        load_slabs(r)

        def fire(b, i):
            for v in range(be // 16):
                sidx[i][pl.ds(16 * v, 16)] = sslab[b, pl.ds(16 * v, 16)]
                didx[i][pl.ds(16 * v, 16)] = dslab[b, pl.ds(16 * v, 16)]
            return [
                pltpu.async_copy(asr_h.at[sidx[i]], ea_s[i], semg[i]),
                pltpu.async_copy(adr_h.at[didx[i]], ea_d[i], semg[i]),
            ]

        def compute(b, i):
            loc0 = (r * bpr + b) * be

            def rows(j, _):
                ve = _edge_alpha_num(ea_s[i], ea_d[i], j, mask8)
                if selm_src is not None:
                    ve = ve * selm_src(loc0 + j)
                exb[i][j] = ve
                return 0

            lax.fori_loop(0, be, rows, 0)

        def sfire(b, i):
            return [pltpu.async_copy(exb[i], sums_sh.at[didx[i]], sems[i],
                                     add=True)]

        _pipelined_blocks(bpr, fire, compute, sfire)
        return 0

    lax.fori_loop(0, nrnd, p1_round, 0)
    plsc.subcore_barrier()

    sums_half = sums_hbm.at[c]
    nfull = RPT // 128
    for i in range(nfull):
        pltpu.sync_copy(sums_sh.at[pl.ds(row0 + i * 128, 128)],
                        sums_half.at[pl.ds(row0 + i * 128, 128)])
    rem = RPT - nfull * 128
    if rem:
        pltpu.sync_copy(sums_sh.at[pl.ds(row0 + RPT - rem, rem)],
                        sums_half.at[pl.ds(row0 + RPT - rem, rem)])
    plsc.subcore_barrier()

    xl_src = xl_h.at[c]
    sums_src = sums_hbm.at[c]

    def p2_round(r, _):
        load_slabs(r)

        def fire(b, i):
            for v in range(be // 16):
                sidx[i][pl.ds(16 * v, 16)] = sslab[b, pl.ds(16 * v, 16)]
                didx[i][pl.ds(16 * v, 16)] = dslab[b, pl.ds(16 * v, 16)]
            return [
                pltpu.async_copy(asr_h.at[sidx[i]], ea_s[i], semg[i]),
                pltpu.async_copy(adr_h.at[didx[i]], ea_d[i], semg[i]),
                pltpu.async_copy(sums_src.at[didx[i]], esum[i], semg[i]),
                pltpu.async_copy(xl_src.at[sidx[i]], msg[i], semg[i]),
            ]

        def compute(b, i):
            loc0 = (r * bpr + b) * be

            def rows(j, _):
                ve = _edge_alpha_num(ea_s[i], ea_d[i], j, mask8)
                if selm_src is not None:
                    ve = ve * selm_src(loc0 + j)
                al = ve / (esum[i][j] + 1e-16)
                alf[pl.ds(pl.multiple_of(j * 16, 16), 16)] = al
                jv = jnp.full((16,), j * 16, i32)
                for v in range(0, 8, 2):
                    hv = c * 4 + (v // 2)
                    am = plsc.load_gather(alf, [jv + hv])
                    sl = pl.ds(16 * v, 16)
                    msg[i][j, sl] = msg[i][j, sl] * am
                    sl2 = pl.ds(16 * (v + 1), 16)
                    msg[i][j, sl2] = msg[i][j, sl2] * am
                return 0

            lax.fori_loop(0, be, rows, 0)

            if write_scores:
                @pl.when(c == 0)
                def _():
                    eb = eb_of(r, b)
                    for j16 in range(be // 16):
                        rv = (jnp.full((16,), j16 * 16, i32) + lane) * 16
                        acc = jnp.zeros((16,), f32)
                        for h in range(8):
                            acc = acc + plsc.load_gather(alf, [rv + h])
                        sc = acc * 0.125
                        eid = jnp.full((16,), eb + j16 * 16, i32) + lane
                        sc = jnp.where(eid < NE1, sc, -1.0)
                        scr[pl.ds(j16 * 16, 16)] = sc
                    pltpu.sync_copy(scr, scores_out.at[pl.ds(eb, be)])

        def sfire(b, i):
            return [pltpu.async_copy(msg[i], acc_sh.at[didx[i]], sems[i],
                                     add=True)]

        _pipelined_blocks(bpr, fire, compute, sfire)
        return 0

    lax.fori_loop(0, nrnd, p2_round, 0)
    plsc.subcore_barrier()


def _write_out_half(msgs_out, acc_sh, c, row0, chunk):
    out_half = msgs_out.at[c]
    nfull = RPT // chunk
    rem = RPT - nfull * chunk
    for i in range(nfull):
        pltpu.sync_copy(acc_sh.at[pl.ds(row0 + i * chunk, chunk)],
                        out_half.at[pl.ds(row0 + i * chunk, chunk)])
    if rem:
        pltpu.sync_copy(acc_sh.at[pl.ds(row0 + RPT - rem, rem)],
                        out_half.at[pl.ds(row0 + RPT - rem, rem)])


def _sc1_body(src2d_h, dst2d_h, asr_h, adr_h, xl_h,
              scores_out, msgs_out, sums_out,
              sslab, dslab, sidxA, sidxB, didxA, didxB,
              ea_sA, ea_sB, ea_dA, ea_dB,
              exbA, exbB, esumA, esumB, alf, msgA, msgB, scr,
              acc_sh, sums_sh, semgA, semgB, semsA, semsB):
    c = lax.axis_index("c")
    s = lax.axis_index("s")
    lane = lax.iota(i32, 16)
    mask8 = lane < 8
    row0 = pl.multiple_of(s * RPT, 8)

    _zero_vmem(msgA, BE1, 128)
    _zero_shared_stripe(msgA, BE1, acc_sh, row0, RPT)
    _zero_vmem(exbA, BE1, 16)
    _zero_shared_stripe(exbA, BE1, sums_sh, row0, RPT)
    plsc.subcore_barrier()

    _sc_conv(BE1, BPR1, NRND1, src2d_h, dst2d_h, asr_h, adr_h,
             xl_h, None, True, scores_out, sums_out, row0,
             sslab, dslab, [sidxA, sidxB], [didxA, didxB],
             [ea_sA, ea_sB], [ea_dA, ea_dB],
             [exbA, exbB], [esumA, esumB], alf, [msgA, msgB], scr,
             sums_sh, acc_sh, [semgA, semgB], [semsA, semsB],
             c, s, mask8, lane)

    _write_out_half(msgs_out, acc_sh, c, row0, BE1)


def _sc2_body(src2d_h, dst2d_h, asr_h, adr_h, xl_h, scores2_h,
              msgs_out, sums_out,
              sslab, dslab, sidxA, sidxB, didxA, didxB,
              ea_sA, ea_sB, ea_dA, ea_dB,
              exbA, exbB, esumA, esumB, alf, msgA, msgB,
              keys, cb, stb,
              acc_sh, sums_sh, stage_sh, stage2_sh,
              semgA, semgB, semsA, semsB):
    c = lax.axis_index("c")
    s = lax.axis_index("s")
    lane = lax.iota(i32, 16)
    mask8 = lane < 8
    row0 = pl.multiple_of(s * RPT, 8)
    base0 = pl.multiple_of(s * CPT2, 8)

    _zero_vmem(msgA, BE2, 128)
    _zero_shared_stripe(msgA, BE2, acc_sh, row0, RPT)
    _zero_vmem(exbA, BE2, 16)
    _zero_shared_stripe(exbA, BE2, sums_sh, row0, RPT)
    pltpu.sync_copy(scores2_h.at[pl.ds(base0, CPT2)], keys)
    plsc.subcore_barrier()

    nv = CPT2 // 16

    def count_gt(thr):
        def cbody(v, acc):
            kf = keys[pl.ds(16 * v, 16)]
            ki = plsc.bitcast(kf, i32)
            return acc + jnp.where(ki > thr, 1, 0).astype(i32)
        acc = lax.fori_loop(0, nv, cbody, jnp.zeros((16,), i32))
        return jnp.sum(acc)

    def stage_scalar(val, sh):
        cb[...] = jnp.full((16,), val, i32)
        pltpu.sync_copy(cb, sh.at[s])

    def read_total(sh):
        pltpu.sync_copy(sh, stb)
        tot = jnp.zeros((16,), i32)
        for r in range(16):
            tot = tot + stb[r]
        return tot[0]

    def wcond(carry):
        lo, hi = carry
        return hi - lo > 1

    def wbody(carry):
        lo, hi = carry
        mid = (lo + hi) // 2
        cnt = count_gt(mid)
        stage_scalar(cnt, stage_sh)
        plsc.subcore_barrier()
        tot = read_total(stage_sh)
        plsc.subcore_barrier()
        pred = tot >= KSEL
        return (jnp.where(pred, mid, lo), jnp.where(pred, hi, mid))

    lo0 = jnp.asarray(-2, i32)
    hi0 = jnp.asarray(1 << 30, i32)
    _, thr = lax.while_loop(wcond, wbody, (lo0, hi0))

    def gq_body(v, acc):
        g, q = acc
        kf = keys[pl.ds(16 * v, 16)]
        ki = plsc.bitcast(kf, i32)
        g = g + jnp.where(ki > thr, 1, 0).astype(i32)
        q = q + jnp.where(ki == thr, 1, 0).astype(i32)
        return (g, q)

    gv, qv = lax.fori_loop(0, nv, gq_body,
                           (jnp.zeros((16,), i32), jnp.zeros((16,), i32)))
    stage_scalar(jnp.sum(gv), stage_sh)
    stage_scalar(jnp.sum(qv), stage2_sh)
    plsc.subcore_barrier()
    g_tot = read_total(stage_sh)
    pltpu.sync_copy(stage2_sh, stb)
    pref = jnp.asarray(0, i32)
    for r in range(16):
        qr = stb[r][0]
        pref = pref + jnp.where(r < s, qr, 0)
    plsc.subcore_barrier()
    need = KSEL - g_tot

    def sel_body(v, cnt):
        kf = keys[pl.ds(16 * v, 16)]
        ki = plsc.bitcast(kf, i32)
        gt = ki > thr
        eq = ki == thr
        eqi = jnp.where(eq, 1, 0).astype(i32)
        pre = plsc.cumsum(eqi)
        rank = jnp.full((16,), cnt, i32) + pre - 1
        keep = jnp.logical_or(gt, jnp.logical_and(eq, rank < need))
        keys[pl.ds(16 * v, 16)] = jnp.where(keep, 1.0, 0.0)
        return cnt + jnp.sum(eqi)

    lax.fori_loop(0, nv, sel_body, pref)
    plsc.subcore_barrier()

    def selv(loc):
        return plsc.load_gather(keys, [jnp.full((16,), loc, i32)])

    _sc_conv(BE2, BPR2, NRND2, src2d_h, dst2d_h, asr_h, adr_h,
             xl_h, selv, False, None, sums_out, row0,
             sslab, dslab, [sidxA, sidxB], [didxA, didxB],
             [ea_sA, ea_sB], [ea_dA, ea_dB],
             [exbA, exbB], [esumA, esumB], alf, [msgA, msgB], None,
             sums_sh, acc_sh, [semgA, semgB], [semsA, semsB],
             c, s, mask8, lane)

    _write_out_half(msgs_out, acc_sh, c, row0, BE2)


_SC_MESH = plsc.VectorSubcoreMesh(core_axis_name="c", subcore_axis_name="s")


def _conv_scratch(be, bpr):
    return [
        pltpu.VMEM((bpr, be), i32),
        pltpu.VMEM((bpr, be), i32),
        pltpu.VMEM((be,), i32),
        pltpu.VMEM((be,), i32),
        pltpu.VMEM((be,), i32),
        pltpu.VMEM((be,), i32),
        pltpu.VMEM((be, 16), f32),
        pltpu.VMEM((be, 16), f32),
        pltpu.VMEM((be, 16), f32),
        pltpu.VMEM((be, 16), f32),
        pltpu.VMEM((be, 16), f32),
        pltpu.VMEM((be, 16), f32),
        pltpu.VMEM((be, 16), f32),
        pltpu.VMEM((be, 16), f32),
        pltpu.VMEM((be * 16,), f32),
        pltpu.VMEM((be, 128), f32),
        pltpu.VMEM((be, 128), f32),
    ]


_SC_PARAMS = pltpu.CompilerParams(needs_layout_passes=False,
                                  use_tc_tiling_on_sc=False)

_sc1_call = pl.kernel(
    _sc1_body,
    out_type=(jax.ShapeDtypeStruct((EP1,), f32),
              jax.ShapeDtypeStruct((2, NP, 128), f32),
              jax.ShapeDtypeStruct((2, NP, 16), f32)),
    mesh=_SC_MESH,
    compiler_params=_SC_PARAMS,
    scratch_types=_conv_scratch(BE1, BPR1) + [
        pltpu.VMEM((BE1,), f32),
        pltpu.VMEM_SHARED((NP, 128), f32),
        pltpu.VMEM_SHARED((NP, 16), f32),
        pltpu.SemaphoreType.DMA,
        pltpu.SemaphoreType.DMA,
        pltpu.SemaphoreType.DMA,
        pltpu.SemaphoreType.DMA,
    ],
)

_sc2_call = pl.kernel(
    _sc2_body,
    out_type=(jax.ShapeDtypeStruct((2, NP, 128), f32),
              jax.ShapeDtypeStruct((2, NP, 16), f32)),
    mesh=_SC_MESH,
    compiler_params=_SC_PARAMS,
    scratch_types=_conv_scratch(BE2, BPR2) + [
        pltpu.VMEM((CPT2,), f32),
        pltpu.VMEM((16,), i32),
        pltpu.VMEM((16, 16), i32),
        pltpu.VMEM_SHARED((NP, 128), f32),
        pltpu.VMEM_SHARED((NP, 16), f32),
        pltpu.VMEM_SHARED((16, 16), i32),
        pltpu.VMEM_SHARED((16, 16), i32),
        pltpu.SemaphoreType.DMA,
        pltpu.SemaphoreType.DMA,
        pltpu.SemaphoreType.DMA,
        pltpu.SemaphoreType.DMA,
    ],
)


def _att_mat(att_s, att_d):
    fl_s = att_s.reshape(HID)
    fl_d = att_d.reshape(HID)
    fidx = jnp.arange(HID)
    head = fidx // HEAD_DIM
    m = jnp.zeros((HID, 32), f32)
    m = m.at[fidx, head].set(fl_s)
    m = m.at[fidx, 16 + head].set(fl_d)
    return m


def kernel(x, edge_index, res_W, res_b, c1_W, c1_as, c1_ad, c1_b,
           bn1_g, bn1_b, bn1_m, bn1_v, c2_W, c2_as, c2_ad, c2_b,
           bn2_g, bn2_b, bn2_m, bn2_v, f1_W, f1_b,
           fbn_g, fbn_b, fbn_m, fbn_v, f2_W, f2_b):
    xpad = jnp.zeros((NP, IN_SIZE), f32).at[:N].set(x)

    loops = jnp.arange(N, dtype=i32)
    src0 = edge_index[0].astype(i32)
    dst0 = edge_index[1].astype(i32)
    padN = jnp.full((EP1 - NE1,), N, i32)
    src1 = jnp.concatenate([src0, loops, padN])
    dst1 = jnp.concatenate([dst0, loops, padN])
    padN2 = jnp.full((EP2 - NE2,), N, i32)
    src2 = jnp.concatenate([src0, loops, loops, padN2])
    dst2 = jnp.concatenate([dst0, loops, loops, padN2])

    am1 = _att_mat(c1_as, c1_ad)
    am2 = _att_mat(c2_as, c2_ad)

    xp, xl1, as1, ad1 = pl.pallas_call(
        _tc1_body,
        grid=(NBLK,),
        in_specs=[_row_spec(IN_SIZE), _full_spec((IN_SIZE, HID)),
                  _full_spec((1, HID)), _full_spec((HID, HID)),
                  _full_spec((HID, 32))],
        out_specs=[_row_spec(HID),
                   pl.BlockSpec((2, RB, 128), lambda i: (0, i, 0)),
                   _row_spec(16), _row_spec(16)],
        out_shape=[jax.ShapeDtypeStruct((NP, HID), f32),
                   jax.ShapeDtypeStruct((2, NP, 128), f32),
                   jax.ShapeDtypeStruct((NP, 16), f32),
                   jax.ShapeDtypeStruct((NP, 16), f32)],
    )(xpad, res_W, res_b.reshape(1, HID), c1_W, am1)

    scores1, msgs1, _ = _sc1_call(src1.reshape(EP1 // BE1, BE1),
                                  dst1.reshape(EP1 // BE1, BE1),
                                  as1, ad1, xl1)

    pv1 = jnp.stack([c1_b, bn1_g, bn1_b, bn1_m, bn1_v])
    xl2, as2, ad2 = pl.pallas_call(
        _tc2_body,
        grid=(NBLK,),
        in_specs=[pl.BlockSpec((2, RB, 128), lambda i: (0, i, 0)),
                  _full_spec((5, HID)), _full_spec((HID, HID)),
                  _full_spec((HID, 32))],
        out_specs=[pl.BlockSpec((2, RB, 128), lambda i: (0, i, 0)),
                   _row_spec(16), _row_spec(16)],
        out_shape=[jax.ShapeDtypeStruct((2, NP, 128), f32),
                   jax.ShapeDtypeStruct((NP, 16), f32),
                   jax.ShapeDtypeStruct((NP, 16), f32)],
    )(msgs1, pv1, c2_W, am2)

    scores2 = jnp.concatenate([
        scores1[:NE1],
        jnp.full((N,), 2.0, f32),
        jnp.full((EP2 - NE2,), -1.0, f32)
    ])
    msgs2, _ = _sc2_call(src2.reshape(EP2 // BE2, BE2),
                         dst2.reshape(EP2 // BE2, BE2),
                         as2, ad2, xl2, scores2)

    pv2 = jnp.stack([c2_b, bn2_g, bn2_b, bn2_m, bn2_v])
    f2w_pad = jnp.zeros((HID // 2, 128), f32).at[:, :OUT_SIZE].set(f2_W)
    qv = jnp.stack([f1_b, fbn_g, fbn_b, fbn_m, fbn_v,
                    jnp.zeros((HID // 2,), f32)])
    qv = jnp.zeros((6, 128), f32).at[:, :HID // 2].set(qv)
    qv = qv.at[5, :OUT_SIZE].set(f2_b)
    out = pl.pallas_call(
        _tc3_body,
        grid=(NBLK,),
        in_specs=[pl.BlockSpec((2, RB, 128), lambda i: (0, i, 0)),
                  _full_spec((5, HID)), _row_spec(HID),
                  _full_spec((HID, HID // 2)), _full_spec((6, 128)),
                  _full_spec((HID // 2, 128))],
        out_specs=[_row_spec(128)],
        out_shape=[jax.ShapeDtypeStruct((NP, 128), f32)],
    )(msgs2, pv2, xp, f1_W, qv, f2w_pad)[0]

    return out[:N, :OUT_SIZE]
